# Initial kernel scaffold; baseline (speedup 1.0000x reference)
import jax
import jax.numpy as jnp
from jax.experimental import pallas as pl


def kernel(index, bra_w, ket_w):
    raise NotImplementedError("write your pallas kernel here")


# speedup vs baseline: 1.3582x; 1.3582x over previous
import jax
import jax.numpy as jnp
from jax import lax
from jax.experimental import pallas as pl
from jax.experimental.pallas import tpu as pltpu
from jax.experimental.pallas import tpu_sc as plsc

NUM_ENTITIES = 1000000
HALF = 32
EMBED = 64
ROWS = 16384
FEATS = 26
B_TOTAL = ROWS * FEATS
NC = 2
NS = 16
NW = NC * NS
B_PER_W = B_TOTAL // NW
CHUNK = 128
NBUF = 4
N_CHUNKS = B_PER_W // CHUNK
N_GROUPS = N_CHUNKS // NBUF


def _body(idx_hbm, bra_hbm, ket_hbm, bra_out, ket_out,
          idx_v, bra_tmp, ket_tmp, bra_stg, ket_stg, gsems, wsems):
    wid = lax.axis_index("s") * NC + lax.axis_index("c")
    base = wid * B_PER_W

    zeros16 = jnp.zeros((16,), jnp.float32)

    def zrow(r, c):
        for s in range(NBUF):
            bra_stg[s, r, pl.ds(32, 16)] = zeros16
            bra_stg[s, r, pl.ds(48, 16)] = zeros16
            ket_stg[s, r, pl.ds(0, 16)] = zeros16
            ket_stg[s, r, pl.ds(16, 16)] = zeros16
        return c

    lax.fori_loop(0, CHUNK, zrow, 0)

    def drain(s, start):
        rows = pl.ds(start, CHUNK)
        pltpu.make_async_copy(
            bra_stg.at[s], bra_out.at[rows], wsems.at[s]
        ).wait()
        pltpu.make_async_copy(
            ket_stg.at[s], ket_out.at[rows], wsems.at[s]
        ).wait()

    def group(g, c):
        for s in range(NBUF):
            i = g * NBUF + s
            start = base + i * CHUNK
            rows = pl.ds(start, CHUNK)

            @pl.when(g >= 1)
            def _(s=s, start=start):
                drain(s, start)

            pltpu.sync_copy(idx_hbm.at[pl.ds(start, CHUNK)], idx_v.at[s])
            cb = pltpu.async_copy(bra_hbm.at[idx_v.at[s]], bra_tmp.at[s],
                                  gsems.at[s])
            ck = pltpu.async_copy(ket_hbm.at[idx_v.at[s]], ket_tmp.at[s],
                                  gsems.at[s])
            cb.wait()
            ck.wait()

            def copy_rows(r4, c, s=s):
                for u in range(4):
                    r = r4 * 4 + u
                    for h in range(2):
                        col = pl.ds(h * 16, 16)
                        bra_stg[s, r, col] = bra_tmp[s, r, col]
                        kcol = pl.ds(HALF + h * 16, 16)
                        ket_stg[s, r, kcol] = ket_tmp[s, r, col]
                return c

            lax.fori_loop(0, CHUNK // 4, copy_rows, 0)
            pltpu.async_copy(bra_stg.at[s], bra_out.at[rows], wsems.at[s])
            pltpu.async_copy(ket_stg.at[s], ket_out.at[rows], wsems.at[s])
        return c

    lax.fori_loop(0, N_GROUPS, group, 0)

    for s in range(NBUF):
        i = (N_GROUPS - 1) * NBUF + s
        drain(s, base + i * CHUNK)


@jax.jit
def _run(index_flat, bra_w, ket_w):
    mesh = plsc.VectorSubcoreMesh(core_axis_name="c", subcore_axis_name="s")
    out = pl.kernel(
        _body,
        out_type=(
            jax.ShapeDtypeStruct((B_TOTAL, EMBED), jnp.float32),
            jax.ShapeDtypeStruct((B_TOTAL, EMBED), jnp.float32),
        ),
        mesh=mesh,
        compiler_params=pltpu.CompilerParams(use_tc_tiling_on_sc=False),
        scratch_types=[
            pltpu.VMEM((NBUF, CHUNK), jnp.int32),
            pltpu.VMEM((NBUF, CHUNK, HALF), jnp.float32),
            pltpu.VMEM((NBUF, CHUNK, HALF), jnp.float32),
            pltpu.VMEM((NBUF, CHUNK, EMBED), jnp.float32),
            pltpu.VMEM((NBUF, CHUNK, EMBED), jnp.float32),
            pltpu.SemaphoreType.DMA((NBUF,)),
            pltpu.SemaphoreType.DMA((NBUF,)),
        ],
    )(index_flat, bra_w, ket_w)
    return out


def kernel(index, bra_w, ket_w):
    index_flat = index.reshape(-1).astype(jnp.int32)
    bra_full, ket_full = _run(index_flat, bra_w, ket_w)
    return (
        bra_full.reshape(ROWS, FEATS, EMBED),
        ket_full.reshape(ROWS, FEATS, EMBED),
    )


# speedup vs baseline: 1.5851x; 1.1670x over previous
import jax
import jax.numpy as jnp
from jax import lax
from jax.experimental import pallas as pl
from jax.experimental.pallas import tpu as pltpu
from jax.experimental.pallas import tpu_sc as plsc

NUM_ENTITIES = 1000000
HALF = 32
EMBED = 64
ROWS = 16384
FEATS = 26
B_TOTAL = ROWS * FEATS
NC = 2
NS = 16
NW = NC * NS
B_PER_W = B_TOTAL // NW
CHUNK = 128
NBUF = 4
N_CHUNKS = B_PER_W // CHUNK
N_GROUPS = N_CHUNKS // NBUF
ZROWS = 1024
N_ZDMA = B_PER_W // ZROWS


def _body(idx_hbm, bra_hbm, ket_hbm, bra_out, ket_out,
          idx_v, bra_tmp, ket_tmp, zero_v, gsems, wsems, zsem):
    wid = lax.axis_index("s") * NC + lax.axis_index("c")
    base = wid * B_PER_W

    zeros16 = jnp.zeros((16,), jnp.float32)

    pltpu.sync_copy(idx_hbm.at[wid], idx_v)

    def zrow(r, c):
        zero_v[r, pl.ds(0, 16)] = zeros16
        zero_v[r, pl.ds(16, 16)] = zeros16
        return c

    lax.fori_loop(0, ZROWS, zrow, 0)

    def zfire(j, c):
        rows = pl.ds(base + j * ZROWS, ZROWS)
        pltpu.async_copy(zero_v, bra_out.at[rows, pl.ds(HALF, HALF)], zsem)
        pltpu.async_copy(zero_v, ket_out.at[rows, pl.ds(0, HALF)], zsem)
        return c

    lax.fori_loop(0, N_ZDMA, zfire, 0)

    def drain(s, start):
        rows = pl.ds(start, CHUNK)
        pltpu.make_async_copy(
            bra_tmp.at[s], bra_out.at[rows, pl.ds(0, HALF)], wsems.at[s]
        ).wait()
        pltpu.make_async_copy(
            ket_tmp.at[s], ket_out.at[rows, pl.ds(HALF, HALF)], wsems.at[s]
        ).wait()

    def group(g, c):
        for s in range(NBUF):
            i = g * NBUF + s

            @pl.when(g >= 1)
            def _(s=s, i=i):
                drain(s, base + (i - NBUF) * CHUNK)

            pltpu.async_copy(bra_hbm.at[idx_v.at[i]], bra_tmp.at[s],
                             gsems.at[s])
            pltpu.async_copy(ket_hbm.at[idx_v.at[i]], ket_tmp.at[s],
                             gsems.at[s])

        for s in range(NBUF):
            i = g * NBUF + s
            rows = pl.ds(base + i * CHUNK, CHUNK)
            pltpu.make_async_copy(
                bra_hbm.at[idx_v.at[i]], bra_tmp.at[s], gsems.at[s]).wait()
            pltpu.make_async_copy(
                ket_hbm.at[idx_v.at[i]], ket_tmp.at[s], gsems.at[s]).wait()
            pltpu.async_copy(bra_tmp.at[s], bra_out.at[rows, pl.ds(0, HALF)],
                             wsems.at[s])
            pltpu.async_copy(ket_tmp.at[s], ket_out.at[rows, pl.ds(HALF, HALF)],
                             wsems.at[s])
        return c

    lax.fori_loop(0, N_GROUPS, group, 0)

    for s in range(NBUF):
        i = (N_GROUPS - 1) * NBUF + s
        drain(s, base + i * CHUNK)

    def zdrain(j, c):
        rows = pl.ds(base + j * ZROWS, ZROWS)
        pltpu.make_async_copy(
            zero_v, bra_out.at[rows, pl.ds(HALF, HALF)], zsem).wait()
        pltpu.make_async_copy(
            zero_v, ket_out.at[rows, pl.ds(0, HALF)], zsem).wait()
        return c

    lax.fori_loop(0, N_ZDMA, zdrain, 0)


@jax.jit
def _run(index_grp, bra_w, ket_w):
    mesh = plsc.VectorSubcoreMesh(core_axis_name="c", subcore_axis_name="s")
    out = pl.kernel(
        _body,
        out_type=(
            jax.ShapeDtypeStruct((B_TOTAL, EMBED), jnp.float32),
            jax.ShapeDtypeStruct((B_TOTAL, EMBED), jnp.float32),
        ),
        mesh=mesh,
        compiler_params=pltpu.CompilerParams(use_tc_tiling_on_sc=False),
        scratch_types=[
            pltpu.VMEM((N_CHUNKS, CHUNK), jnp.int32),
            pltpu.VMEM((NBUF, CHUNK, HALF), jnp.float32),
            pltpu.VMEM((NBUF, CHUNK, HALF), jnp.float32),
            pltpu.VMEM((ZROWS, HALF), jnp.float32),
            pltpu.SemaphoreType.DMA((NBUF,)),
            pltpu.SemaphoreType.DMA((NBUF,)),
            pltpu.SemaphoreType.DMA,
        ],
    )(index_grp, bra_w, ket_w)
    return out


def kernel(index, bra_w, ket_w):
    index_grp = index.reshape(NW, N_CHUNKS, CHUNK).astype(jnp.int32)
    bra_full, ket_full = _run(index_grp, bra_w, ket_w)
    return (
        bra_full.reshape(ROWS, FEATS, EMBED),
        ket_full.reshape(ROWS, FEATS, EMBED),
    )
